# baseline (speedup 1.0000x reference)
import jax
import jax.numpy as jnp
from jax import lax
from jax.experimental import pallas as pl
from jax.experimental.pallas import tpu as pltpu
from jax.experimental.pallas import tpu_sc as plsc

N_NODES = 100000
N_EDGES = 6400000
SIGMA = 1.0
EPSILON = 5.0
ALPHA = 5.0
CUTOFF = 2.5

NPAD = 100096
RT = NPAD // 16
SCHUNK = 2048
SREM = RT - 3 * SCHUNK

C = 2048
JB = 16
CH = 98
EPT = C * CH
NE_PAD = 32 * EPT
BPT = EPT // 128


def _rsqrt(d2):
    i = lax.bitcast_convert_type(d2, jnp.int32)
    y = lax.bitcast_convert_type(jnp.int32(0x5F3759DF) - (i >> 1), jnp.float32)
    for _ in range(3):
        y = y * (1.5 - 0.5 * d2 * y * y)
    return y


def _sc_kernel(px_hbm, py_hbm, pz_hbm, src_hbm, dst_hbm, zeros_hbm,
               out_ae, out_fx, out_fy, out_fz, out_ep,
               px_sh, py_sh, pz_sh, ae_sh, fx_sh, fy_sh, fz_sh,
               isA, idA, isB, idB, is_s, id_s,
               gsxA, gsyA, gszA, gdxA, gdyA, gdzA,
               gsxB, gsyB, gszB, gdxB, gdyB, gdzB,
               heb, fxb, fyb, fzb, nfxb, nfyb, nfzb,
               stage, ebuf, semgA, semgB, sems):
    cid = lax.axis_index("c")
    sid = lax.axis_index("s")
    tl = cid * 16 + sid
    iota = lax.iota(jnp.int32, 16)
    base = sid * RT

    pltpu.sync_copy(zeros_hbm, stage)
    for acc in (ae_sh, fx_sh, fy_sh, fz_sh):
        for k in range(3):
            pltpu.sync_copy(stage, acc.at[pl.ds(base + k * SCHUNK, SCHUNK)])
        pltpu.sync_copy(stage.at[pl.ds(0, SREM)],
                        acc.at[pl.ds(base + 3 * SCHUNK, SREM)])
    for src, dstt in ((px_hbm, px_sh), (py_hbm, py_sh), (pz_hbm, pz_sh)):
        for k in range(4):
            r0 = base + k * SCHUNK
            cnt = SCHUNK if k < 3 else SREM
            pltpu.sync_copy(src.at[pl.ds(r0, cnt)], stage.at[pl.ds(0, cnt)])
            pltpu.sync_copy(stage.at[pl.ds(0, cnt)], dstt.at[pl.ds(r0, cnt)])
    plsc.subcore_barrier()


    def load_idx(c, isb, idb):
        e0 = tl * EPT + c * C
        pltpu.sync_copy(src_hbm.at[pl.ds(e0, C)], isb)
        pltpu.sync_copy(dst_hbm.at[pl.ds(e0, C)], idb)

    def fire_gathers(isb, idb, gb, sg):
        gsx, gsy, gsz, gdx, gdy, gdz = gb
        pltpu.async_copy(px_sh.at[isb], gsx, sg)
        pltpu.async_copy(py_sh.at[isb], gsy, sg)
        pltpu.async_copy(pz_sh.at[isb], gsz, sg)
        pltpu.async_copy(px_sh.at[idb], gdx, sg)
        pltpu.async_copy(py_sh.at[idb], gdy, sg)
        pltpu.async_copy(pz_sh.at[idb], gdz, sg)

    def drain_gathers(isb, idb, gb, sg):
        gsx, gsy, gsz, gdx, gdy, gdz = gb
        pltpu.make_async_copy(px_sh.at[isb], gsx, sg).wait()
        pltpu.make_async_copy(py_sh.at[isb], gsy, sg).wait()
        pltpu.make_async_copy(pz_sh.at[isb], gsz, sg).wait()
        pltpu.make_async_copy(px_sh.at[idb], gdx, sg).wait()
        pltpu.make_async_copy(py_sh.at[idb], gdy, sg).wait()
        pltpu.make_async_copy(pz_sh.at[idb], gdz, sg).wait()

    def fire_scatters():
        pltpu.async_copy(heb, ae_sh.at[is_s], sems, add=True)
        pltpu.async_copy(heb, ae_sh.at[id_s], sems, add=True)
        pltpu.async_copy(nfxb, fx_sh.at[is_s], sems, add=True)
        pltpu.async_copy(fxb, fx_sh.at[id_s], sems, add=True)
        pltpu.async_copy(nfyb, fy_sh.at[is_s], sems, add=True)
        pltpu.async_copy(fyb, fy_sh.at[id_s], sems, add=True)
        pltpu.async_copy(nfzb, fz_sh.at[is_s], sems, add=True)
        pltpu.async_copy(fzb, fz_sh.at[id_s], sems, add=True)

    def drain_scatters():
        pltpu.make_async_copy(heb, ae_sh.at[is_s], sems).wait()
        pltpu.make_async_copy(heb, ae_sh.at[id_s], sems).wait()
        pltpu.make_async_copy(nfxb, fx_sh.at[is_s], sems).wait()
        pltpu.make_async_copy(fxb, fx_sh.at[id_s], sems).wait()
        pltpu.make_async_copy(nfyb, fy_sh.at[is_s], sems).wait()
        pltpu.make_async_copy(fyb, fy_sh.at[id_s], sems).wait()
        pltpu.make_async_copy(nfzb, fz_sh.at[is_s], sems).wait()
        pltpu.make_async_copy(fzb, fz_sh.at[id_s], sems).wait()

    def compute_chunk(gb, isb, idb, evec):
        gsx, gsy, gsz, gdx, gdy, gdz = gb

        def step(st, ev):
            w = pl.ds(st * 16, 16)
            is_s[w] = isb[w]
            id_s[w] = idb[w]
            dx = gdx[w] - gsx[w]
            dy = gdy[w] - gsy[w]
            dz = gdz[w] - gsz[w]
            d2 = dx * dx + dy * dy + dz * dz
            y = _rsqrt(d2)
            dr = d2 * y
            mask = (d2 > 0.0) & (d2 < CUTOFF * CUTOFF)
            ex = jnp.exp(-ALPHA * (dr - SIGMA))
            om = 1.0 - ex
            pe = jnp.where(mask, EPSILON * om * om - EPSILON, 0.0)
            fs = jnp.where(mask, (-2.0 * ALPHA * EPSILON) * ex * om, 0.0) * y
            fvx = fs * dx
            fvy = fs * dy
            fvz = fs * dz
            heb[w] = 0.5 * pe
            fxb[w] = fvx
            fyb[w] = fvy
            fzb[w] = fvz
            nfxb[w] = -fvx
            nfyb[w] = -fvy
            nfzb[w] = -fvz
            return ev + pe

        return lax.fori_loop(0, C // 16, step, evec)

    gA = (gsxA, gsyA, gszA, gdxA, gdyA, gdzA)
    gB = (gsxB, gsyB, gszB, gdxB, gdyB, gdzB)

    load_idx(0, isA, idA)
    fire_gathers(isA, idA, gA, semgA)

    def pipe_body(t, evec):
        load_idx(2 * t + 1, isB, idB)
        fire_gathers(isB, idB, gB, semgB)

        @pl.when(t > 0)
        def _():
            drain_scatters()

        drain_gathers(isA, idA, gA, semgA)
        evec = compute_chunk(gA, isA, idA, evec)
        fire_scatters()

        @pl.when(t < CH // 2 - 1)
        def _():
            load_idx(2 * t + 2, isA, idA)
            fire_gathers(isA, idA, gA, semgA)

        drain_scatters()
        drain_gathers(isB, idB, gB, semgB)
        evec = compute_chunk(gB, isB, idB, evec)
        fire_scatters()
        return evec

    evec = lax.fori_loop(0, CH // 2, pipe_body, jnp.zeros((16,), jnp.float32))
    drain_scatters()
    ebuf[...] = evec
    pltpu.sync_copy(ebuf, out_ep.at[pl.ds(tl * 16, 16)])
    plsc.subcore_barrier()

    for acc, out in ((ae_sh, out_ae), (fx_sh, out_fx),
                     (fy_sh, out_fy), (fz_sh, out_fz)):
        for k in range(4):
            r0 = base + k * SCHUNK
            cnt = SCHUNK if k < 3 else SREM
            pltpu.sync_copy(acc.at[pl.ds(r0, cnt)], stage.at[pl.ds(0, cnt)])
            pltpu.sync_copy(stage.at[pl.ds(0, cnt)],
                            out.at[pl.ds(cid * NPAD + r0, cnt)])


def _combine_body(ae_ref, fx_ref, fy_ref, fz_ref, ep_ref,
                  ae_out, f_out, e_out):
    ae_out[0, :] = ae_ref[pl.ds(0, NPAD)] + ae_ref[pl.ds(NPAD, NPAD)]
    f_out[0, :] = fx_ref[pl.ds(0, NPAD)] + fx_ref[pl.ds(NPAD, NPAD)]
    f_out[1, :] = fy_ref[pl.ds(0, NPAD)] + fy_ref[pl.ds(NPAD, NPAD)]
    f_out[2, :] = fz_ref[pl.ds(0, NPAD)] + fz_ref[pl.ds(NPAD, NPAD)]
    e_out[...] = (0.5 * jnp.sum(ep_ref[...])).reshape(1, 1)


@jax.jit
def kernel(positions, mapping):
    f32 = jnp.float32
    pad = jnp.zeros((NPAD - N_NODES,), f32)
    px = jnp.concatenate([positions[:, 0], pad])
    py = jnp.concatenate([positions[:, 1], pad])
    pz = jnp.concatenate([positions[:, 2], pad])

    pad_idx = (jnp.arange(NE_PAD - N_EDGES, dtype=jnp.int32) % N_NODES)
    src = jnp.concatenate([mapping[0], pad_idx])
    dst = jnp.concatenate([mapping[1], pad_idx])
    zeros = jnp.zeros((SCHUNK,), f32)

    mesh = plsc.VectorSubcoreMesh(core_axis_name="c", subcore_axis_name="s")
    sc = pl.kernel(
        _sc_kernel,
        out_type=[
            jax.ShapeDtypeStruct((2 * NPAD,), f32),
            jax.ShapeDtypeStruct((2 * NPAD,), f32),
            jax.ShapeDtypeStruct((2 * NPAD,), f32),
            jax.ShapeDtypeStruct((2 * NPAD,), f32),
            jax.ShapeDtypeStruct((512,), f32),
        ],
        mesh=mesh,
        scratch_types=[
            pltpu.VMEM_SHARED((NPAD,), f32),
            pltpu.VMEM_SHARED((NPAD,), f32),
            pltpu.VMEM_SHARED((NPAD,), f32),
            pltpu.VMEM_SHARED((NPAD,), f32),
            pltpu.VMEM_SHARED((NPAD,), f32),
            pltpu.VMEM_SHARED((NPAD,), f32),
            pltpu.VMEM_SHARED((NPAD,), f32),
            pltpu.VMEM((C,), jnp.int32),
            pltpu.VMEM((C,), jnp.int32),
            pltpu.VMEM((C,), jnp.int32),
            pltpu.VMEM((C,), jnp.int32),
            pltpu.VMEM((C,), jnp.int32),
            pltpu.VMEM((C,), jnp.int32),
            pltpu.VMEM((C,), f32),
            pltpu.VMEM((C,), f32),
            pltpu.VMEM((C,), f32),
            pltpu.VMEM((C,), f32),
            pltpu.VMEM((C,), f32),
            pltpu.VMEM((C,), f32),
            pltpu.VMEM((C,), f32),
            pltpu.VMEM((C,), f32),
            pltpu.VMEM((C,), f32),
            pltpu.VMEM((C,), f32),
            pltpu.VMEM((C,), f32),
            pltpu.VMEM((C,), f32),
            pltpu.VMEM((C,), f32),
            pltpu.VMEM((C,), f32),
            pltpu.VMEM((C,), f32),
            pltpu.VMEM((C,), f32),
            pltpu.VMEM((C,), f32),
            pltpu.VMEM((C,), f32),
            pltpu.VMEM((C,), f32),
            pltpu.VMEM((SCHUNK,), f32),
            pltpu.VMEM((16,), f32),
            pltpu.SemaphoreType.DMA,
            pltpu.SemaphoreType.DMA,
            pltpu.SemaphoreType.DMA,
        ],
    )
    ae2, fx2, fy2, fz2, ep = sc(px, py, pz, src, dst, zeros)

    ae_c, f_c, e_c = pl.pallas_call(
        _combine_body,
        out_shape=[
            jax.ShapeDtypeStruct((1, NPAD), f32),
            jax.ShapeDtypeStruct((3, NPAD), f32),
            jax.ShapeDtypeStruct((1, 1), f32),
        ],
    )(ae2, fx2, fy2, fz2, ep)

    energy = e_c[0, 0]
    atom_energies = ae_c[0, :N_NODES]
    forces = f_c[:, :N_NODES].T
    return energy, atom_energies, forces


# speedup vs baseline: 1.3281x; 1.3281x over previous
import jax
import jax.numpy as jnp
from jax import lax
from jax.experimental import pallas as pl
from jax.experimental.pallas import tpu as pltpu
from jax.experimental.pallas import tpu_sc as plsc

N_NODES = 100000
N_EDGES = 6400000
SIGMA = 1.0
EPSILON = 5.0
ALPHA = 5.0
CUTOFF = 2.5

NPAD = 100096
RT = NPAD // 16
SCHUNK = 2048
SREM = RT - 3 * SCHUNK

C = 4096
CH = 49
EPT = C * CH
NE_PAD = 32 * EPT
BPT = EPT // 128


def _rsqrt(d2):
    i = lax.bitcast_convert_type(d2, jnp.int32)
    y = lax.bitcast_convert_type(jnp.int32(0x5F3759DF) - (i >> 1), jnp.float32)
    for _ in range(2):
        y = y * (1.5 - 0.5 * d2 * y * y)
    return y


def _sc_kernel(px_hbm, py_hbm, pz_hbm, src_hbm, dst_hbm, zeros_hbm,
               out_ae, out_fx, out_fy, out_fz, out_ep,
               px_sh, py_sh, pz_sh, ae_sh, fx_sh, fy_sh, fz_sh,
               is1d, id1d, gsx, gsy, gsz, gdx, gdy, gdz,
               heb, fxb, fyb, fzb, nfxb, nfyb, nfzb,
               stage, ebuf, semg, sems):
    cid = lax.axis_index("c")
    sid = lax.axis_index("s")
    tl = cid * 16 + sid
    iota = lax.iota(jnp.int32, 16)
    base = sid * RT

    pltpu.sync_copy(zeros_hbm, stage)
    for acc in (ae_sh, fx_sh, fy_sh, fz_sh):
        for k in range(3):
            pltpu.sync_copy(stage, acc.at[pl.ds(base + k * SCHUNK, SCHUNK)])
        pltpu.sync_copy(stage.at[pl.ds(0, SREM)],
                        acc.at[pl.ds(base + 3 * SCHUNK, SREM)])
    for src, dstt in ((px_hbm, px_sh), (py_hbm, py_sh), (pz_hbm, pz_sh)):
        for k in range(4):
            r0 = base + k * SCHUNK
            cnt = SCHUNK if k < 3 else SREM
            pltpu.sync_copy(src.at[pl.ds(r0, cnt)], stage.at[pl.ds(0, cnt)])
            pltpu.sync_copy(stage.at[pl.ds(0, cnt)], dstt.at[pl.ds(r0, cnt)])
    plsc.subcore_barrier()

    def chunk_body(c, evec):
        e0 = tl * EPT + c * C
        pltpu.sync_copy(src_hbm.at[pl.ds(e0, C)], is1d)
        pltpu.sync_copy(dst_hbm.at[pl.ds(e0, C)], id1d)
        gds = [
            pltpu.async_copy(px_sh.at[is1d], gsx, semg),
            pltpu.async_copy(py_sh.at[is1d], gsy, semg),
            pltpu.async_copy(pz_sh.at[is1d], gsz, semg),
            pltpu.async_copy(px_sh.at[id1d], gdx, semg),
            pltpu.async_copy(py_sh.at[id1d], gdy, semg),
            pltpu.async_copy(pz_sh.at[id1d], gdz, semg),
        ]
        for d in gds:
            d.wait()

        def step(s, ev):
            w = pl.ds(s * 16, 16)
            dx = gdx[w] - gsx[w]
            dy = gdy[w] - gsy[w]
            dz = gdz[w] - gsz[w]
            d2 = dx * dx + dy * dy + dz * dz
            y = _rsqrt(d2)
            dr = d2 * y
            mask = (d2 > 0.0) & (d2 < CUTOFF * CUTOFF)
            ex = jnp.exp(-ALPHA * (dr - SIGMA))
            om = 1.0 - ex
            pe = jnp.where(mask, EPSILON * om * om - EPSILON, 0.0)
            fs = jnp.where(mask, (-2.0 * ALPHA * EPSILON) * ex * om, 0.0) * y
            fvx = fs * dx
            fvy = fs * dy
            fvz = fs * dz
            heb[w] = 0.5 * pe
            fxb[w] = fvx
            fyb[w] = fvy
            fzb[w] = fvz
            nfxb[w] = -fvx
            nfyb[w] = -fvy
            nfzb[w] = -fvz
            return ev + pe

        evec = lax.fori_loop(0, C // 16, step, evec)
        sds = [
            pltpu.async_copy(heb, ae_sh.at[is1d], sems, add=True),
            pltpu.async_copy(heb, ae_sh.at[id1d], sems, add=True),
            pltpu.async_copy(nfxb, fx_sh.at[is1d], sems, add=True),
            pltpu.async_copy(fxb, fx_sh.at[id1d], sems, add=True),
            pltpu.async_copy(nfyb, fy_sh.at[is1d], sems, add=True),
            pltpu.async_copy(fyb, fy_sh.at[id1d], sems, add=True),
            pltpu.async_copy(nfzb, fz_sh.at[is1d], sems, add=True),
            pltpu.async_copy(fzb, fz_sh.at[id1d], sems, add=True),
        ]
        for d in sds:
            d.wait()
        return evec

    evec = lax.fori_loop(0, CH, chunk_body, jnp.zeros((16,), jnp.float32))
    ebuf[...] = evec
    pltpu.sync_copy(ebuf, out_ep.at[pl.ds(tl * 16, 16)])
    plsc.subcore_barrier()

    for acc, out in ((ae_sh, out_ae), (fx_sh, out_fx),
                     (fy_sh, out_fy), (fz_sh, out_fz)):
        for k in range(4):
            r0 = base + k * SCHUNK
            cnt = SCHUNK if k < 3 else SREM
            pltpu.sync_copy(acc.at[pl.ds(r0, cnt)], stage.at[pl.ds(0, cnt)])
            pltpu.sync_copy(stage.at[pl.ds(0, cnt)],
                            out.at[pl.ds(cid * NPAD + r0, cnt)])


def _combine_body(ae_ref, fx_ref, fy_ref, fz_ref, ep_ref,
                  ae_out, f_out, e_out):
    ae_out[0, :] = ae_ref[pl.ds(0, NPAD)] + ae_ref[pl.ds(NPAD, NPAD)]
    f_out[0, :] = fx_ref[pl.ds(0, NPAD)] + fx_ref[pl.ds(NPAD, NPAD)]
    f_out[1, :] = fy_ref[pl.ds(0, NPAD)] + fy_ref[pl.ds(NPAD, NPAD)]
    f_out[2, :] = fz_ref[pl.ds(0, NPAD)] + fz_ref[pl.ds(NPAD, NPAD)]
    e_out[...] = (0.5 * jnp.sum(ep_ref[...])).reshape(1, 1)


@jax.jit
def kernel(positions, mapping):
    f32 = jnp.float32
    pad = jnp.zeros((NPAD - N_NODES,), f32)
    px = jnp.concatenate([positions[:, 0], pad])
    py = jnp.concatenate([positions[:, 1], pad])
    pz = jnp.concatenate([positions[:, 2], pad])

    pad_idx = (jnp.arange(NE_PAD - N_EDGES, dtype=jnp.int32) % N_NODES)
    src = jnp.concatenate([mapping[0], pad_idx])
    dst = jnp.concatenate([mapping[1], pad_idx])
    zeros = jnp.zeros((SCHUNK,), f32)

    mesh = plsc.VectorSubcoreMesh(core_axis_name="c", subcore_axis_name="s")
    sc = pl.kernel(
        _sc_kernel,
        out_type=[
            jax.ShapeDtypeStruct((2 * NPAD,), f32),
            jax.ShapeDtypeStruct((2 * NPAD,), f32),
            jax.ShapeDtypeStruct((2 * NPAD,), f32),
            jax.ShapeDtypeStruct((2 * NPAD,), f32),
            jax.ShapeDtypeStruct((512,), f32),
        ],
        mesh=mesh,
        scratch_types=[
            pltpu.VMEM_SHARED((NPAD,), f32),
            pltpu.VMEM_SHARED((NPAD,), f32),
            pltpu.VMEM_SHARED((NPAD,), f32),
            pltpu.VMEM_SHARED((NPAD,), f32),
            pltpu.VMEM_SHARED((NPAD,), f32),
            pltpu.VMEM_SHARED((NPAD,), f32),
            pltpu.VMEM_SHARED((NPAD,), f32),
            pltpu.VMEM((C,), jnp.int32),
            pltpu.VMEM((C,), jnp.int32),
            pltpu.VMEM((C,), f32),
            pltpu.VMEM((C,), f32),
            pltpu.VMEM((C,), f32),
            pltpu.VMEM((C,), f32),
            pltpu.VMEM((C,), f32),
            pltpu.VMEM((C,), f32),
            pltpu.VMEM((C,), f32),
            pltpu.VMEM((C,), f32),
            pltpu.VMEM((C,), f32),
            pltpu.VMEM((C,), f32),
            pltpu.VMEM((C,), f32),
            pltpu.VMEM((C,), f32),
            pltpu.VMEM((C,), f32),
            pltpu.VMEM((SCHUNK,), f32),
            pltpu.VMEM((16,), f32),
            pltpu.SemaphoreType.DMA,
            pltpu.SemaphoreType.DMA,
        ],
    )
    ae2, fx2, fy2, fz2, ep = sc(px, py, pz, src, dst, zeros)

    ae_c, f_c, e_c = pl.pallas_call(
        _combine_body,
        out_shape=[
            jax.ShapeDtypeStruct((1, NPAD), f32),
            jax.ShapeDtypeStruct((3, NPAD), f32),
            jax.ShapeDtypeStruct((1, 1), f32),
        ],
    )(ae2, fx2, fy2, fz2, ep)

    energy = e_c[0, 0]
    atom_energies = ae_c[0, :N_NODES]
    forces = f_c[:, :N_NODES].T
    return energy, atom_energies, forces


# speedup vs baseline: 1.3698x; 1.0314x over previous
import jax
import jax.numpy as jnp
from jax import lax
from jax.experimental import pallas as pl
from jax.experimental.pallas import tpu as pltpu
from jax.experimental.pallas import tpu_sc as plsc

N_NODES = 100000
N_EDGES = 6400000
SIGMA = 1.0
EPSILON = 5.0
ALPHA = 5.0
CUTOFF = 2.5

NPAD = 100096
RT = NPAD // 16
SCHUNK = 2048
SREM = RT - 3 * SCHUNK

C = 4000
CH = 50
EPT = C * CH


def _rsqrt(d2):
    i = lax.bitcast_convert_type(d2, jnp.int32)
    y = lax.bitcast_convert_type(jnp.int32(0x5F3759DF) - (i >> 1), jnp.float32)
    for _ in range(2):
        y = y * (1.5 - 0.5 * d2 * y * y)
    return y


def _sc_kernel(px_hbm, py_hbm, pz_hbm, src_hbm, dst_hbm, zeros_hbm,
               out_ae, out_fx, out_fy, out_fz, out_ep,
               px_sh, py_sh, pz_sh, ae_sh, fx_sh, fy_sh, fz_sh,
               is1d, id1d, gsx, gsy, gsz, gdx, gdy, gdz,
               heb, fxb, fyb, fzb, nfxb, nfyb, nfzb,
               stage, ebuf, semg, sems):
    cid = lax.axis_index("c")
    sid = lax.axis_index("s")
    tl = cid * 16 + sid
    iota = lax.iota(jnp.int32, 16)
    base = sid * RT

    pltpu.sync_copy(zeros_hbm, stage)
    for acc in (ae_sh, fx_sh, fy_sh, fz_sh):
        for k in range(3):
            pltpu.sync_copy(stage, acc.at[pl.ds(base + k * SCHUNK, SCHUNK)])
        pltpu.sync_copy(stage.at[pl.ds(0, SREM)],
                        acc.at[pl.ds(base + 3 * SCHUNK, SREM)])
    for src, dstt in ((px_hbm, px_sh), (py_hbm, py_sh), (pz_hbm, pz_sh)):
        for k in range(4):
            r0 = base + k * SCHUNK
            cnt = SCHUNK if k < 3 else SREM
            pltpu.sync_copy(src.at[pl.ds(r0, cnt)], stage.at[pl.ds(0, cnt)])
            pltpu.sync_copy(stage.at[pl.ds(0, cnt)], dstt.at[pl.ds(r0, cnt)])
    plsc.subcore_barrier()

    def chunk_body(c, evec):
        e0 = tl * EPT + c * C
        pltpu.sync_copy(src_hbm.at[pl.ds(e0, C)], is1d)
        pltpu.sync_copy(dst_hbm.at[pl.ds(e0, C)], id1d)
        gds = [
            pltpu.async_copy(px_sh.at[is1d], gsx, semg),
            pltpu.async_copy(py_sh.at[is1d], gsy, semg),
            pltpu.async_copy(pz_sh.at[is1d], gsz, semg),
            pltpu.async_copy(px_sh.at[id1d], gdx, semg),
            pltpu.async_copy(py_sh.at[id1d], gdy, semg),
            pltpu.async_copy(pz_sh.at[id1d], gdz, semg),
        ]
        for d in gds:
            d.wait()

        def step(s, ev):
            w = pl.ds(s * 16, 16)
            dx = gdx[w] - gsx[w]
            dy = gdy[w] - gsy[w]
            dz = gdz[w] - gsz[w]
            d2 = dx * dx + dy * dy + dz * dz
            y = _rsqrt(d2)
            dr = d2 * y
            mask = (d2 > 0.0) & (d2 < CUTOFF * CUTOFF)
            ex = jnp.exp(-ALPHA * (dr - SIGMA))
            om = 1.0 - ex
            pe = jnp.where(mask, EPSILON * om * om - EPSILON, 0.0)
            fs = jnp.where(mask, (-2.0 * ALPHA * EPSILON) * ex * om, 0.0) * y
            fvx = fs * dx
            fvy = fs * dy
            fvz = fs * dz
            heb[w] = 0.5 * pe
            fxb[w] = fvx
            fyb[w] = fvy
            fzb[w] = fvz
            nfxb[w] = -fvx
            nfyb[w] = -fvy
            nfzb[w] = -fvz
            return ev + pe

        evec = lax.fori_loop(0, C // 16, step, evec)
        sds = [
            pltpu.async_copy(heb, ae_sh.at[is1d], sems, add=True),
            pltpu.async_copy(heb, ae_sh.at[id1d], sems, add=True),
            pltpu.async_copy(nfxb, fx_sh.at[is1d], sems, add=True),
            pltpu.async_copy(fxb, fx_sh.at[id1d], sems, add=True),
            pltpu.async_copy(nfyb, fy_sh.at[is1d], sems, add=True),
            pltpu.async_copy(fyb, fy_sh.at[id1d], sems, add=True),
            pltpu.async_copy(nfzb, fz_sh.at[is1d], sems, add=True),
            pltpu.async_copy(fzb, fz_sh.at[id1d], sems, add=True),
        ]
        for d in sds:
            d.wait()
        return evec

    evec = lax.fori_loop(0, CH, chunk_body, jnp.zeros((16,), jnp.float32))
    ebuf[...] = evec
    pltpu.sync_copy(ebuf, out_ep.at[pl.ds(tl * 16, 16)])
    plsc.subcore_barrier()

    for acc, out in ((ae_sh, out_ae), (fx_sh, out_fx),
                     (fy_sh, out_fy), (fz_sh, out_fz)):
        for k in range(4):
            r0 = base + k * SCHUNK
            cnt = SCHUNK if k < 3 else SREM
            pltpu.sync_copy(acc.at[pl.ds(r0, cnt)], stage.at[pl.ds(0, cnt)])
            pltpu.sync_copy(stage.at[pl.ds(0, cnt)],
                            out.at[pl.ds(cid * NPAD + r0, cnt)])


def _combine_body(ae_ref, fx_ref, fy_ref, fz_ref, ep_ref,
                  ae_out, f_out, e_out):
    ae_out[0, :] = ae_ref[pl.ds(0, NPAD)] + ae_ref[pl.ds(NPAD, NPAD)]
    f_out[0, :] = fx_ref[pl.ds(0, NPAD)] + fx_ref[pl.ds(NPAD, NPAD)]
    f_out[1, :] = fy_ref[pl.ds(0, NPAD)] + fy_ref[pl.ds(NPAD, NPAD)]
    f_out[2, :] = fz_ref[pl.ds(0, NPAD)] + fz_ref[pl.ds(NPAD, NPAD)]
    e_out[...] = (0.5 * jnp.sum(ep_ref[...])).reshape(1, 1)


@jax.jit
def kernel(positions, mapping):
    f32 = jnp.float32
    pad = jnp.zeros((NPAD - N_NODES,), f32)
    px = jnp.concatenate([positions[:, 0], pad])
    py = jnp.concatenate([positions[:, 1], pad])
    pz = jnp.concatenate([positions[:, 2], pad])

    src = mapping[0]
    dst = mapping[1]
    zeros = jnp.zeros((SCHUNK,), f32)

    mesh = plsc.VectorSubcoreMesh(core_axis_name="c", subcore_axis_name="s")
    sc = pl.kernel(
        _sc_kernel,
        out_type=[
            jax.ShapeDtypeStruct((2 * NPAD,), f32),
            jax.ShapeDtypeStruct((2 * NPAD,), f32),
            jax.ShapeDtypeStruct((2 * NPAD,), f32),
            jax.ShapeDtypeStruct((2 * NPAD,), f32),
            jax.ShapeDtypeStruct((512,), f32),
        ],
        mesh=mesh,
        scratch_types=[
            pltpu.VMEM_SHARED((NPAD,), f32),
            pltpu.VMEM_SHARED((NPAD,), f32),
            pltpu.VMEM_SHARED((NPAD,), f32),
            pltpu.VMEM_SHARED((NPAD,), f32),
            pltpu.VMEM_SHARED((NPAD,), f32),
            pltpu.VMEM_SHARED((NPAD,), f32),
            pltpu.VMEM_SHARED((NPAD,), f32),
            pltpu.VMEM((C,), jnp.int32),
            pltpu.VMEM((C,), jnp.int32),
            pltpu.VMEM((C,), f32),
            pltpu.VMEM((C,), f32),
            pltpu.VMEM((C,), f32),
            pltpu.VMEM((C,), f32),
            pltpu.VMEM((C,), f32),
            pltpu.VMEM((C,), f32),
            pltpu.VMEM((C,), f32),
            pltpu.VMEM((C,), f32),
            pltpu.VMEM((C,), f32),
            pltpu.VMEM((C,), f32),
            pltpu.VMEM((C,), f32),
            pltpu.VMEM((C,), f32),
            pltpu.VMEM((C,), f32),
            pltpu.VMEM((SCHUNK,), f32),
            pltpu.VMEM((16,), f32),
            pltpu.SemaphoreType.DMA,
            pltpu.SemaphoreType.DMA,
        ],
    )
    ae2, fx2, fy2, fz2, ep = sc(px, py, pz, src, dst, zeros)

    ae_c, f_c, e_c = pl.pallas_call(
        _combine_body,
        out_shape=[
            jax.ShapeDtypeStruct((1, NPAD), f32),
            jax.ShapeDtypeStruct((3, NPAD), f32),
            jax.ShapeDtypeStruct((1, 1), f32),
        ],
    )(ae2, fx2, fy2, fz2, ep)

    energy = e_c[0, 0]
    atom_energies = ae_c[0, :N_NODES]
    forces = f_c[:, :N_NODES].T
    return energy, atom_energies, forces
